# baseline (speedup 1.0000x reference)
import functools

import jax
import jax.numpy as jnp
from jax.experimental import pallas as pl
from jax.experimental.pallas import tpu as pltpu

_LAYERS = 3


def _diffusion_kernel(emb_ref, g_ref, out_ref, buf_ref, acc_ref, *, bm, layers,
                      precision):
    l = pl.program_id(0)
    i = pl.program_id(1)

    @pl.when(jnp.logical_and(l == 0, i == 0))
    def _init():
        buf_ref[0] = emb_ref[...]
        acc_ref[...] = emb_ref[...]

    x = buf_ref[l % 2]
    y = jax.lax.dot_general(
        g_ref[...], x, (((1,), (0,)), ((), ())),
        preferred_element_type=jnp.float32, precision=precision)
    buf_ref[(l + 1) % 2, pl.ds(i * bm, bm), :] = y
    new_acc = acc_ref[pl.ds(i * bm, bm), :] + y
    acc_ref[pl.ds(i * bm, bm), :] = new_acc
    out_ref[...] = new_acc


@jax.jit
def kernel(embedding, graph):
    n, d = embedding.shape
    bm = 200
    assert n % bm == 0
    grid = (_LAYERS, n // bm)
    body = functools.partial(_diffusion_kernel, bm=bm, layers=_LAYERS,
                             precision=jax.lax.Precision.HIGHEST)
    return pl.pallas_call(
        body,
        grid=grid,
        in_specs=[
            pl.BlockSpec((n, d), lambda l, i: (0, 0)),
            pl.BlockSpec((bm, n), lambda l, i: (i, 0)),
        ],
        out_specs=pl.BlockSpec((bm, d), lambda l, i: (i, 0)),
        out_shape=jax.ShapeDtypeStruct((n, d), jnp.float32),
        scratch_shapes=[
            pltpu.VMEM((2, n, d), jnp.float32),
            pltpu.VMEM((n, d), jnp.float32),
        ],
    )(embedding, graph)


# speedup vs baseline: 2.7154x; 2.7154x over previous
import functools

import jax
import jax.numpy as jnp
from jax.experimental import pallas as pl
from jax.experimental.pallas import tpu as pltpu

_LAYERS = 3


def _diffusion_kernel(emb_ref, g_ref, out_ref, buf_ref, acc_ref, *, bm, layers,
                      precision):
    l = pl.program_id(0)
    i = pl.program_id(1)

    @pl.when(jnp.logical_and(l == 0, i == 0))
    def _init():
        buf_ref[0] = emb_ref[...]
        acc_ref[...] = emb_ref[...]

    x = buf_ref[l % 2]
    y = jax.lax.dot_general(
        g_ref[...], x, (((1,), (0,)), ((), ())),
        preferred_element_type=jnp.float32, precision=precision)
    buf_ref[(l + 1) % 2, pl.ds(i * bm, bm), :] = y
    new_acc = acc_ref[pl.ds(i * bm, bm), :] + y
    acc_ref[pl.ds(i * bm, bm), :] = new_acc
    out_ref[...] = new_acc


@jax.jit
def kernel(embedding, graph):
    n, d = embedding.shape
    bm = 200
    assert n % bm == 0
    grid = (_LAYERS, n // bm)
    body = functools.partial(_diffusion_kernel, bm=bm, layers=_LAYERS,
                             precision=jax.lax.Precision.DEFAULT)
    return pl.pallas_call(
        body,
        grid=grid,
        in_specs=[
            pl.BlockSpec((n, d), lambda l, i: (0, 0)),
            pl.BlockSpec((bm, n), lambda l, i: (i, 0)),
        ],
        out_specs=pl.BlockSpec((bm, d), lambda l, i: (i, 0)),
        out_shape=jax.ShapeDtypeStruct((n, d), jnp.float32),
        scratch_shapes=[
            pltpu.VMEM((2, n, d), jnp.float32),
            pltpu.VMEM((n, d), jnp.float32),
        ],
    )(embedding, graph)


# speedup vs baseline: 2.9555x; 1.0884x over previous
import functools

import jax
import jax.numpy as jnp
from jax.experimental import pallas as pl
from jax.experimental.pallas import tpu as pltpu


def _layer1_and_cast_kernel(emb_ref, g_ref, g16_ref, y1_ref):
    g = g_ref[...]
    g16 = g.astype(jnp.bfloat16)
    g16_ref[...] = g16
    y1_ref[...] = jax.lax.dot_general(
        g16, emb_ref[...].astype(jnp.bfloat16), (((1,), (0,)), ((), ())),
        preferred_element_type=jnp.float32)


def _layers23_kernel(emb_ref, y1_ref, g16_ref, out_ref, buf_ref, acc_ref, *,
                     bm):
    l = pl.program_id(0)
    i = pl.program_id(1)

    @pl.when(jnp.logical_and(l == 0, i == 0))
    def _init():
        buf_ref[0] = y1_ref[...].astype(jnp.bfloat16)
        acc_ref[...] = emb_ref[...] + y1_ref[...]

    y = jax.lax.dot_general(
        g16_ref[...], buf_ref[l % 2], (((1,), (0,)), ((), ())),
        preferred_element_type=jnp.float32)
    buf_ref[(l + 1) % 2, pl.ds(i * bm, bm), :] = y.astype(jnp.bfloat16)
    new_acc = acc_ref[pl.ds(i * bm, bm), :] + y
    acc_ref[pl.ds(i * bm, bm), :] = new_acc
    out_ref[...] = new_acc


@jax.jit
def kernel(embedding, graph):
    n, d = embedding.shape
    bm_a = 400
    bm_b = 400
    assert n % bm_a == 0 and n % bm_b == 0

    g16, y1 = pl.pallas_call(
        _layer1_and_cast_kernel,
        grid=(n // bm_a,),
        in_specs=[
            pl.BlockSpec((n, d), lambda i: (0, 0)),
            pl.BlockSpec((bm_a, n), lambda i: (i, 0)),
        ],
        out_specs=[
            pl.BlockSpec((bm_a, n), lambda i: (i, 0)),
            pl.BlockSpec((bm_a, d), lambda i: (i, 0)),
        ],
        out_shape=[
            jax.ShapeDtypeStruct((n, n), jnp.bfloat16),
            jax.ShapeDtypeStruct((n, d), jnp.float32),
        ],
    )(embedding, graph)

    return pl.pallas_call(
        functools.partial(_layers23_kernel, bm=bm_b),
        grid=(2, n // bm_b),
        in_specs=[
            pl.BlockSpec((n, d), lambda l, i: (0, 0)),
            pl.BlockSpec((n, d), lambda l, i: (0, 0)),
            pl.BlockSpec((bm_b, n), lambda l, i: (i, 0)),
        ],
        out_specs=pl.BlockSpec((bm_b, d), lambda l, i: (i, 0)),
        out_shape=jax.ShapeDtypeStruct((n, d), jnp.float32),
        scratch_shapes=[
            pltpu.VMEM((2, n, d), jnp.bfloat16),
            pltpu.VMEM((n, d), jnp.float32),
        ],
    )(embedding, y1, g16)
